# baseline (device time: 12142 ns/iter reference)
import os

import jax
import jax.numpy as jnp
from jax import lax
from jax.experimental import pallas as pl
from jax.experimental.pallas import tpu as pltpu

Z = 4
ROWS = 512
COLS = 256
B = 160

ABLATE = os.environ.get("ABLATE", "")


def kernel(x, dest):
    dest2 = dest.reshape(1, ROWS).astype(jnp.int32)

    def body(x_ref, d_ref, out_ref, seg_ref, yg_ref, dg_ref,
             sxs, sxr, sds, sdr):
        me_x = lax.axis_index("x")
        me_y = lax.axis_index("y")
        me_z = lax.axis_index("z")

        if ABLATE != "pure":
            bar = pltpu.get_barrier_semaphore()
            for dz in range(1, Z):
                pl.semaphore_signal(
                    bar, inc=1,
                    device_id=(me_x, me_y, lax.rem(me_z + dz, Z)),
                    device_id_type=pl.DeviceIdType.MESH,
                )
            pl.semaphore_wait(bar, Z - 1)

        dg_ref[me_z] = d_ref[...]

        d_sends = []
        for k, dz in enumerate(range(1, Z)) if ABLATE not in ("nocomm", "pure") else []:
            p = lax.rem(me_z + dz, Z)
            rd = pltpu.make_async_remote_copy(
                src_ref=dg_ref.at[me_z],
                dst_ref=dg_ref.at[me_z],
                send_sem=sds.at[k],
                recv_sem=sdr.at[k],
                device_id=(me_x, me_y, p),
                device_id_type=pl.DeviceIdType.MESH,
            )
            rd.start()
            d_sends.append(rd)

        d_loc = d_ref[...]
        x_bf = x_ref[...].astype(jnp.bfloat16)
        if ABLATE != "nocompute":
            r_iota = lax.broadcasted_iota(jnp.int32, (Z, ROWS), 0)
            m_all = (r_iota == d_loc).astype(jnp.int32)
            tri = (
                lax.broadcasted_iota(jnp.int32, (ROWS, ROWS), 0)
                <= lax.broadcasted_iota(jnp.int32, (ROWS, ROWS), 1)
            ).astype(jnp.float32)
            cs_all = lax.dot_general(
                m_all.astype(jnp.float32), tri, (((1,), (0,)), ((), ())),
                preferred_element_type=jnp.float32,
            ).astype(jnp.int32)
            kk = jnp.sum(m_all * cs_all, axis=0, keepdims=True)
            key_flat = d_loc * B + kk - 1
            bi = lax.broadcasted_iota(jnp.int32, (Z * B, ROWS), 0)
            p_all = (bi == key_flat).astype(jnp.bfloat16)
            seg_ref[...] = lax.dot_general(
                p_all, x_bf, (((1,), (0,)), ((), ())),
                preferred_element_type=jnp.float32,
            ).astype(jnp.bfloat16)
        else:
            seg_ref[...] = jnp.zeros((Z * B, COLS), jnp.bfloat16)

        yg_ref[pl.ds(me_z * B, B), :] = seg_ref[pl.ds(me_z * B, B), :]
        x_sends = []
        for k, dz in enumerate(range(1, Z)) if ABLATE not in ("nocomm", "pure") else []:
            p = lax.rem(me_z + dz, Z)
            rx = pltpu.make_async_remote_copy(
                src_ref=seg_ref.at[pl.ds(p * B, B), :],
                dst_ref=yg_ref.at[pl.ds(me_z * B, B), :],
                send_sem=sxs.at[k],
                recv_sem=sxr.at[k],
                device_id=(me_x, me_y, p),
                device_id_type=pl.DeviceIdType.MESH,
            )
            rx.start()
            x_sends.append(rx)

        for dz in range(1, Z) if ABLATE not in ("nocomm", "pure") else []:
            s = lax.rem(me_z + dz, Z)
            ks = Z - dz - 1
            pltpu.make_async_remote_copy(
                src_ref=dg_ref.at[s], dst_ref=dg_ref.at[s],
                send_sem=sds.at[0], recv_sem=sdr.at[ks],
                device_id=(me_x, me_y, s),
                device_id_type=pl.DeviceIdType.MESH,
            ).wait_recv()
            pltpu.make_async_remote_copy(
                src_ref=yg_ref.at[pl.ds(s * B, B), :],
                dst_ref=yg_ref.at[pl.ds(s * B, B), :],
                send_sem=sxs.at[0], recv_sem=sxr.at[ks],
                device_id=(me_x, me_y, s),
                device_id_type=pl.DeviceIdType.MESH,
            ).wait_recv()

        if ABLATE != "nocompute":
            oi = lax.broadcasted_iota(jnp.int32, (ROWS, B), 0)
            ji = lax.broadcasted_iota(jnp.int32, (ROWS, B), 1)
            acc = jnp.zeros((ROWS, COLS), jnp.float32)
            offset = jnp.int32(0)
            for c in range(Z):
                cnt = jnp.sum((dg_ref[c] == me_z).astype(jnp.int32))
                q = (oi == ji + offset).astype(jnp.bfloat16)
                acc = acc + lax.dot_general(
                    q, yg_ref[pl.ds(c * B, B), :], (((1,), (0,)), ((), ())),
                    preferred_element_type=jnp.float32,
                )
                offset = offset + cnt
            out_ref[...] = acc
        else:
            for c in range(Z):
                out_ref[pl.ds(c * 128, 128), :] = (
                    yg_ref[pl.ds(c * B, 128), :].astype(jnp.float32)
                )

        for rd in d_sends:
            rd.wait_send()
        for rx in x_sends:
            rx.wait_send()

    return pl.pallas_call(
        body,
        out_shape=jax.ShapeDtypeStruct((ROWS, COLS), jnp.float32),
        in_specs=[
            pl.BlockSpec(memory_space=pltpu.VMEM),
            pl.BlockSpec(memory_space=pltpu.VMEM),
        ],
        out_specs=pl.BlockSpec(memory_space=pltpu.VMEM),
        scratch_shapes=[
            pltpu.VMEM((Z * B, COLS), jnp.bfloat16),
            pltpu.VMEM((Z * B, COLS), jnp.bfloat16),
            pltpu.VMEM((Z, 1, ROWS), jnp.int32),
            pltpu.SemaphoreType.DMA((Z - 1,)),
            pltpu.SemaphoreType.DMA((Z - 1,)),
            pltpu.SemaphoreType.DMA((Z - 1,)),
            pltpu.SemaphoreType.DMA((Z - 1,)),
        ],
        compiler_params=(
            pltpu.CompilerParams()
            if ABLATE == "pure"
            else pltpu.CompilerParams(collective_id=0)
        ),
    )(x, dest2)


# device time: 11324 ns/iter; 1.0722x vs baseline; 1.0722x over previous
import os

import jax
import jax.numpy as jnp
from jax import lax
from jax.experimental import pallas as pl
from jax.experimental.pallas import tpu as pltpu

Z = 4
ROWS = 512
COLS = 256
B = 160

ABLATE = os.environ.get("ABLATE", "")
_COMM = ABLATE not in ("nocomm", "pure", "trivial")


def kernel(x, dest):
    dest2 = dest.reshape(1, ROWS).astype(jnp.int32)

    def body(x_ref, d_ref, out_ref, seg_ref, yg_ref, dg_ref,
             sxs, sxr, sds, sdr):
        me_x = lax.axis_index("x")
        me_y = lax.axis_index("y")
        me_z = lax.axis_index("z")

        if ABLATE == "trivial":
            out_ref[...] = x_ref[...]
            return

        if ABLATE not in ("pure", "trivial"):
            bar = pltpu.get_barrier_semaphore()
            for dz in range(1, Z):
                pl.semaphore_signal(
                    bar, inc=1,
                    device_id=(me_x, me_y, lax.rem(me_z + dz, Z)),
                    device_id_type=pl.DeviceIdType.MESH,
                )

        dg_ref[me_z] = d_ref[...]

        d_loc = d_ref[...]
        x_bf = x_ref[...].astype(jnp.bfloat16)
        r_iota = lax.broadcasted_iota(jnp.int32, (Z, ROWS), 0)
        m_all = (r_iota == d_loc).astype(jnp.int32)
        tri = (
            lax.broadcasted_iota(jnp.int32, (ROWS, ROWS), 0)
            <= lax.broadcasted_iota(jnp.int32, (ROWS, ROWS), 1)
        ).astype(jnp.float32)
        cs_all = lax.dot_general(
            m_all.astype(jnp.float32), tri, (((1,), (0,)), ((), ())),
            preferred_element_type=jnp.float32,
        ).astype(jnp.int32)
        seg_iota = lax.broadcasted_iota(jnp.int32, (B, ROWS), 0)
        for r in range(Z):
            key = jnp.where(m_all[r:r + 1, :] > 0, cs_all[r:r + 1, :] - 1, -1)
            p_sel = (seg_iota == key).astype(jnp.bfloat16)
            seg_ref[r] = lax.dot_general(
                p_sel, x_bf, (((1,), (0,)), ((), ())),
                preferred_element_type=jnp.float32,
            ).astype(jnp.bfloat16)

        oi = lax.broadcasted_iota(jnp.int32, (ROWS, B), 0)
        ji = lax.broadcasted_iota(jnp.int32, (ROWS, B), 1)

        yg_ref[me_z] = seg_ref[me_z]

        if ABLATE not in ("pure", "trivial"):
            pl.semaphore_wait(bar, Z - 1)

        d_sends = []
        x_sends = []
        for k, dz in enumerate(range(1, Z)) if _COMM else []:
            p = lax.rem(me_z + dz, Z)
            rd = pltpu.make_async_remote_copy(
                src_ref=dg_ref.at[me_z],
                dst_ref=dg_ref.at[me_z],
                send_sem=sds.at[k],
                recv_sem=sdr.at[k],
                device_id=(me_x, me_y, p),
                device_id_type=pl.DeviceIdType.MESH,
            )
            rd.start()
            d_sends.append(rd)
            rx = pltpu.make_async_remote_copy(
                src_ref=seg_ref.at[p],
                dst_ref=yg_ref.at[me_z],
                send_sem=sxs.at[k],
                recv_sem=sxr.at[k],
                device_id=(me_x, me_y, p),
                device_id_type=pl.DeviceIdType.MESH,
            )
            rx.start()
            x_sends.append(rx)

        for dz in range(1, Z) if _COMM else []:
            s = lax.rem(me_z + dz, Z)
            pltpu.make_async_remote_copy(
                src_ref=dg_ref.at[s], dst_ref=dg_ref.at[s],
                send_sem=sds.at[0], recv_sem=sdr.at[Z - dz - 1],
                device_id=(me_x, me_y, s),
                device_id_type=pl.DeviceIdType.MESH,
            ).wait_recv()

        qs = []
        offset = jnp.int32(0)
        for c in range(Z):
            cnt = jnp.sum((dg_ref[c] == me_z).astype(jnp.int32))
            qs.append((oi == ji + offset).astype(jnp.bfloat16))
            offset = offset + cnt

        for dz in range(1, Z) if _COMM else []:
            s = lax.rem(me_z + dz, Z)
            pltpu.make_async_remote_copy(
                src_ref=yg_ref.at[s], dst_ref=yg_ref.at[s],
                send_sem=sxs.at[0], recv_sem=sxr.at[Z - dz - 1],
                device_id=(me_x, me_y, s),
                device_id_type=pl.DeviceIdType.MESH,
            ).wait_recv()

        acc = jnp.zeros((ROWS, COLS), jnp.float32)
        for c in range(Z):
            acc = acc + lax.dot_general(
                qs[c], yg_ref[c], (((1,), (0,)), ((), ())),
                preferred_element_type=jnp.float32,
            )
        out_ref[...] = acc

        for rd in d_sends:
            rd.wait_send()
        for rx in x_sends:
            rx.wait_send()

    return pl.pallas_call(
        body,
        out_shape=jax.ShapeDtypeStruct((ROWS, COLS), jnp.float32),
        in_specs=[
            pl.BlockSpec(memory_space=pltpu.VMEM),
            pl.BlockSpec(memory_space=pltpu.VMEM),
        ],
        out_specs=pl.BlockSpec(memory_space=pltpu.VMEM),
        scratch_shapes=[
            pltpu.VMEM((Z, B, COLS), jnp.bfloat16),
            pltpu.VMEM((Z, B, COLS), jnp.bfloat16),
            pltpu.VMEM((Z, 1, ROWS), jnp.int32),
            pltpu.SemaphoreType.DMA((Z - 1,)),
            pltpu.SemaphoreType.DMA((Z - 1,)),
            pltpu.SemaphoreType.DMA((Z - 1,)),
            pltpu.SemaphoreType.DMA((Z - 1,)),
        ],
        compiler_params=(
            pltpu.CompilerParams()
            if ABLATE in ("pure", "trivial")
            else pltpu.CompilerParams(collective_id=0)
        ),
    )(x, dest2)


# device time: 11115 ns/iter; 1.0924x vs baseline; 1.0188x over previous
import os

import jax
import jax.numpy as jnp
from jax import lax
from jax.experimental import pallas as pl
from jax.experimental.pallas import tpu as pltpu

Z = 4
ROWS = 512
COLS = 256
B = 160

ABLATE = os.environ.get("ABLATE", "")
_COMM = ABLATE not in ("nocomm", "pure", "trivial")


def kernel(x, dest):
    dest2 = dest.reshape(1, ROWS).astype(jnp.int32)

    def body(x_ref, d_ref, out_ref, seg_ref, yg_ref, dg_ref,
             sxs, sxr, sds, sdr):
        me_x = lax.axis_index("x")
        me_y = lax.axis_index("y")
        me_z = lax.axis_index("z")

        if ABLATE == "trivial":
            out_ref[...] = x_ref[...]
            return

        if ABLATE not in ("pure", "trivial"):
            bar = pltpu.get_barrier_semaphore()
            for dz in range(1, Z):
                pl.semaphore_signal(
                    bar, inc=1,
                    device_id=(me_x, me_y, lax.rem(me_z + dz, Z)),
                    device_id_type=pl.DeviceIdType.MESH,
                )
            if os.environ.get("EARLYWAIT"):
                pl.semaphore_wait(bar, Z - 1)

        dg_ref[me_z] = d_ref[...]

        d_loc = d_ref[...]
        x_bf = x_ref[...].astype(jnp.bfloat16)
        r_iota = lax.broadcasted_iota(jnp.int32, (Z, ROWS), 0)
        m_all = (r_iota == d_loc).astype(jnp.int32)
        tri = (
            lax.broadcasted_iota(jnp.int32, (ROWS, ROWS), 0)
            <= lax.broadcasted_iota(jnp.int32, (ROWS, ROWS), 1)
        ).astype(jnp.float32)
        cs_all = lax.dot_general(
            m_all.astype(jnp.float32), tri, (((1,), (0,)), ((), ())),
            preferred_element_type=jnp.float32,
        ).astype(jnp.int32)
        seg_iota = lax.broadcasted_iota(jnp.int32, (B, ROWS), 0)
        for r in range(Z):
            key = jnp.where(m_all[r:r + 1, :] > 0, cs_all[r:r + 1, :] - 1, -1)
            p_sel = (seg_iota == key).astype(jnp.bfloat16)
            seg_ref[r] = lax.dot_general(
                p_sel, x_bf, (((1,), (0,)), ((), ())),
                preferred_element_type=jnp.float32,
            ).astype(jnp.bfloat16)

        oi = lax.broadcasted_iota(jnp.int32, (ROWS, B), 0)
        ji = lax.broadcasted_iota(jnp.int32, (ROWS, B), 1)

        yg_ref[me_z] = seg_ref[me_z]

        if ABLATE not in ("pure", "trivial") and not os.environ.get("EARLYWAIT"):
            pl.semaphore_wait(bar, Z - 1)

        d_sends = []
        x_sends = []
        for k, dz in enumerate(range(1, Z)) if _COMM else []:
            p = lax.rem(me_z + dz, Z)
            rd = pltpu.make_async_remote_copy(
                src_ref=dg_ref.at[me_z],
                dst_ref=dg_ref.at[me_z],
                send_sem=sds.at[k],
                recv_sem=sdr.at[k],
                device_id=(me_x, me_y, p),
                device_id_type=pl.DeviceIdType.MESH,
            )
            rd.start()
            d_sends.append(rd)
            rx = pltpu.make_async_remote_copy(
                src_ref=seg_ref.at[p],
                dst_ref=yg_ref.at[me_z],
                send_sem=sxs.at[k],
                recv_sem=sxr.at[k],
                device_id=(me_x, me_y, p),
                device_id_type=pl.DeviceIdType.MESH,
            )
            rx.start()
            x_sends.append(rx)

        for dz in range(1, Z) if _COMM else []:
            s = lax.rem(me_z + dz, Z)
            pltpu.make_async_remote_copy(
                src_ref=dg_ref.at[s], dst_ref=dg_ref.at[s],
                send_sem=sds.at[0], recv_sem=sdr.at[Z - dz - 1],
                device_id=(me_x, me_y, s),
                device_id_type=pl.DeviceIdType.MESH,
            ).wait_recv()

        offs = []
        running = jnp.int32(0)
        for c in range(Z):
            offs.append(running)
            running = running + jnp.sum((dg_ref[c] == me_z).astype(jnp.int32))

        def place(s, acc):
            off = jnp.int32(0)
            for c in range(Z):
                off = off + jnp.where(s == c, offs[c], 0)
            q = (oi == ji + off).astype(jnp.bfloat16)
            return acc + lax.dot_general(
                q, yg_ref[s], (((1,), (0,)), ((), ())),
                preferred_element_type=jnp.float32,
            )

        acc = place(me_z, jnp.zeros((ROWS, COLS), jnp.float32))
        for dz in ((1, 3, 2) if _COMM else ()):
            s = lax.rem(me_z + dz, Z)
            pltpu.make_async_remote_copy(
                src_ref=yg_ref.at[s], dst_ref=yg_ref.at[s],
                send_sem=sxs.at[0], recv_sem=sxr.at[Z - dz - 1],
                device_id=(me_x, me_y, s),
                device_id_type=pl.DeviceIdType.MESH,
            ).wait_recv()
            acc = place(s, acc)
        out_ref[...] = acc

        for rd in d_sends:
            rd.wait_send()
        for rx in x_sends:
            rx.wait_send()

    return pl.pallas_call(
        body,
        out_shape=jax.ShapeDtypeStruct((ROWS, COLS), jnp.float32),
        in_specs=[
            pl.BlockSpec(memory_space=pltpu.VMEM),
            pl.BlockSpec(memory_space=pltpu.VMEM),
        ],
        out_specs=pl.BlockSpec(memory_space=pltpu.VMEM),
        scratch_shapes=[
            pltpu.VMEM((Z, B, COLS), jnp.bfloat16),
            pltpu.VMEM((Z, B, COLS), jnp.bfloat16),
            pltpu.VMEM((Z, 1, ROWS), jnp.int32),
            pltpu.SemaphoreType.DMA((Z - 1,)),
            pltpu.SemaphoreType.DMA((Z - 1,)),
            pltpu.SemaphoreType.DMA((Z - 1,)),
            pltpu.SemaphoreType.DMA((Z - 1,)),
        ],
        compiler_params=(
            pltpu.CompilerParams()
            if ABLATE in ("pure", "trivial")
            else pltpu.CompilerParams(collective_id=0)
        ),
    )(x, dest2)
